# baseline (speedup 1.0000x reference)
import functools

import jax
import jax.numpy as jnp
from jax import lax
from jax.experimental import pallas as pl
from jax.experimental.pallas import tpu as pltpu
from jax.experimental.pallas import tpu_sc as plsc

B = 16384
D = 16
L = 16
NC, NS = 2, 16
NW = NC * NS
BPW = B // NW
CHUNK = 128
NCHUNK = BPW // CHUNK
GROUPS = BPW // L

_mesh = plsc.VectorSubcoreMesh(core_axis_name="c", subcore_axis_name="s")


@functools.partial(
    pl.kernel,
    mesh=_mesh,
    compiler_params=pltpu.CompilerParams(needs_layout_passes=False,
                                         use_tc_tiling_on_sc=False),
    out_type=jax.ShapeDtypeStruct((B,), jnp.float32),
    scratch_types=[
        pltpu.VMEM((BPW,), jnp.int32),
        pltpu.VMEM((BPW,), jnp.int32),
        pltpu.VMEM((BPW, D), jnp.float32),
        pltpu.VMEM((BPW, D), jnp.float32),
        pltpu.VMEM((BPW,), jnp.float32),
        pltpu.VMEM((L * D,), jnp.float32),
        pltpu.SemaphoreType.DMA,
        pltpu.SemaphoreType.DMA,
    ],
)
def _sc_dot(user_hbm, song_hbm, uw_hbm, sw_hbm, out_hbm,
            uidx_v, sidx_v, urows_v, srows_v, out_v, prod_v, sem_u, sem_s):
    wid = lax.axis_index("s") * NC + lax.axis_index("c")
    base = wid * BPW

    pltpu.sync_copy(user_hbm.at[pl.ds(base, BPW)], uidx_v)
    pltpu.sync_copy(song_hbm.at[pl.ds(base, BPW)], sidx_v)

    copies = []
    for j in range(NCHUNK):
        sl = pl.ds(j * CHUNK, CHUNK)
        copies.append(pltpu.async_copy(uw_hbm.at[uidx_v.at[sl]],
                                       urows_v.at[sl], sem_u))
        copies.append(pltpu.async_copy(sw_hbm.at[sidx_v.at[sl]],
                                       srows_v.at[sl], sem_s))
    for c in copies:
        c.wait()

    lane = lax.iota(jnp.int32, L)

    def group_body(g, carry):
        base_row = g * L
        for k in range(L):
            u = urows_v[base_row + k, :]
            s = srows_v[base_row + k, :]
            prod_v[pl.ds(k * D, D)] = u * s
        acc = jnp.zeros((L,), jnp.float32)
        for d in range(D):
            acc = acc + plsc.load_gather(prod_v, [lane * D + d])
        out_v[pl.ds(base_row, L)] = jnp.maximum(acc, 0.0)
        return carry

    lax.fori_loop(0, GROUPS, group_body, 0)

    pltpu.sync_copy(out_v, out_hbm.at[pl.ds(base, BPW)])


def kernel(user, song, user_weight, song_weight):
    return _sc_dot(user, song, user_weight, song_weight)
